# baseline (speedup 1.0000x reference)
import functools

import jax
import jax.numpy as jnp
from jax import lax
from jax.experimental import pallas as pl
from jax.experimental.pallas import tpu as pltpu
from jax.experimental.pallas import tpu_sc as plsc

B = 8192
H = 2048
DIN = 3 * H
DOUT = 2048
BM = 512
BN = 1024
NB = DOUT // BN
CAPR = B + 2 * BM
MBLK = CAPR // BM
BF = jnp.bfloat16

NC = 2
NS = 16
NW = NC * NS
CH = 16
SLICE = B // NW

_mesh = plsc.VectorSubcoreMesh(core_axis_name="c", subcore_axis_name="s")


def _leaky(x):
    return jnp.where(x >= 0, x, 0.01 * x)


@functools.partial(
    pl.kernel,
    out_type=[jax.ShapeDtypeStruct((CAPR, H), jnp.float32)] * 3,
    mesh=_mesh,
    compiler_params=pltpu.CompilerParams(needs_layout_passes=False),
    scratch_types=[
        pltpu.VMEM((CH,), jnp.int32),
        pltpu.VMEM((16,), jnp.int32),
        pltpu.VMEM((CH, H), jnp.float32),
        pltpu.VMEM((CH, H), jnp.float32),
        pltpu.SemaphoreType.DMA,
        pltpu.SemaphoreType.DMA,
    ],
)
def _sc_gather(t1, t2, t3, idx, cnt, g1, g2, g3,
               idx_v, cnt_v, ra, rb, sa, sb):
    wid = lax.axis_index("s") * NC + lax.axis_index("c")
    pltpu.sync_copy(cnt, cnt_v)
    cv = cnt_v[...]
    c1s = cv[0]
    c2s = cv[1]
    c1as = cv[2]

    def _chunk(p0):
        p0 = pl.multiple_of(p0, CH)
        pltpu.sync_copy(idx.at[pl.ds(p0, CH)], idx_v)
        cpa = pltpu.async_copy(t1.at[idx_v], ra, sa)
        cpb = pltpu.async_copy(t2.at[idx_v], rb, sb)
        cpa.wait()
        pltpu.sync_copy(ra, g1.at[pl.ds(p0, CH)])
        cpa2 = pltpu.async_copy(t3.at[idx_v], ra, sa)
        cpb.wait()
        pltpu.sync_copy(rb, g2.at[pl.ds(p0, CH)])
        cpa2.wait()
        pltpu.sync_copy(ra, g3.at[pl.ds(p0, CH)])

    c1ch = lax.shift_right_logical(c1s + (CH - 1), 4)
    c2ch = lax.shift_right_logical(c2s + (CH - 1), 4)
    n1w = lax.shift_right_logical(c1ch - wid + (NW - 1), 5)
    n2w = lax.shift_right_logical(c2ch - wid + (NW - 1), 5)

    def _body1(i, carry):
        _chunk((i * NW + wid) * CH)
        return carry

    def _body2(i, carry):
        _chunk(c1as + (i * NW + wid) * CH)
        return carry

    lax.fori_loop(0, n1w, _body1, 0)
    lax.fori_loop(0, n2w, _body2, 0)


@functools.partial(
    pl.kernel,
    out_type=jax.ShapeDtypeStruct((B,), jnp.float32),
    mesh=_mesh,
    compiler_params=pltpu.CompilerParams(needs_layout_passes=False),
    scratch_types=[
        pltpu.VMEM((CAPR,), jnp.float32),
        pltpu.VMEM((SLICE,), jnp.int32),
        pltpu.VMEM((SLICE,), jnp.int32),
        pltpu.VMEM((SLICE,), jnp.float32),
        pltpu.VMEM((SLICE,), jnp.float32),
    ],
)
def _sc_merge(v, pos, sel, bas, res, v_ts, pos_v, sel_v, bas_v, out_v):
    wid = lax.axis_index("s") * NC + lax.axis_index("c")
    o = wid * SLICE
    pltpu.sync_copy(v, v_ts)
    pltpu.sync_copy(pos.at[pl.ds(o, SLICE)], pos_v)
    pltpu.sync_copy(sel.at[pl.ds(o, SLICE)], sel_v)
    pltpu.sync_copy(bas.at[pl.ds(o, SLICE)], bas_v)
    for j in range(SLICE // CH):
        pv = pos_v[pl.ds(j * CH, CH)]
        col = plsc.load_gather(v_ts, [pv])
        sv = sel_v[pl.ds(j * CH, CH)]
        bv = bas_v[pl.ds(j * CH, CH)]
        out_v[pl.ds(j * CH, CH)] = jnp.where(sv > 0, col, bv)
    pltpu.sync_copy(out_v, res.at[pl.ds(o, SLICE)])


def _k_layer1(sc_ref, x1, x2, x3, w, b, out):
    m = pl.program_id(1)

    @pl.when(m < sc_ref[1])
    def _():
        xa = x1[...].astype(BF)
        xb = x2[...].astype(BF)
        xc = x3[...].astype(BF)
        W = w[0]
        acc = jnp.dot(xa, W[0:H], preferred_element_type=jnp.float32)
        acc += jnp.dot(xb, W[H:2 * H], preferred_element_type=jnp.float32)
        acc += jnp.dot(xc, W[2 * H:3 * H], preferred_element_type=jnp.float32)
        acc += b[0]
        out[...] = _leaky(acc).astype(BF)


def _k_layers234(sc_ref, h1, w2, b2, w3, b3, w4, b4, out):
    m = pl.program_id(0)

    @pl.when(m < sc_ref[1])
    def _():
        x = h1[...]
        h2 = _leaky(jnp.dot(x, w2[0], preferred_element_type=jnp.float32)
                    + b2[0]).astype(BF)
        h3 = _leaky(jnp.dot(h2, w3[0], preferred_element_type=jnp.float32)
                    + b3[0])
        col = jnp.sum(h3 * w4[0], axis=1) + b4[0, 0]
        out[...] = col.reshape(1, 1, BM)


def _mclamp(m, sc):
    return jnp.minimum(m, jnp.maximum(sc[1] - 1, 0))


def _lvl(m, sc):
    return jnp.where(_mclamp(m, sc) >= sc[0], 1, 0)


def kernel(l_ty, l_ey, l_y, t,
           W1_l1, b1_l1, W2_l1, b2_l1, W3_l1, b3_l1, W4_l1, b4_l1,
           W1_l2, b1_l2, W2_l2, b2_l2, W3_l2, b3_l2, W4_l2, b4_l2):
    W1s = jnp.stack([W1_l1.T, W1_l2.T]).astype(BF)
    W2s = jnp.stack([W2_l1.T, W2_l2.T]).astype(BF)
    W3s = jnp.stack([W3_l1.T, W3_l2.T]).astype(BF)
    W4s = jnp.stack([W4_l1, W4_l2])
    b1s = jnp.stack([b1_l1, b1_l2])[:, None, :]
    b2s = jnp.stack([b2_l1, b2_l2])[:, None, :]
    b3s = jnp.stack([b3_l1, b3_l2])[:, None, :]
    b4s = jnp.broadcast_to(jnp.stack([b4_l1, b4_l2]).reshape(2, 1, 1),
                           (2, 1, BM))

    tf = t[:, 0]
    m1 = tf == 1
    m2 = tf == 2
    c1 = jnp.sum(m1).astype(jnp.int32)
    c2 = jnp.sum(m2).astype(jnp.int32)
    nb1 = (c1 + BM - 1) // BM
    nbtot = nb1 + (c2 + BM - 1) // BM
    c1a = nb1 * BM
    idx1 = jnp.argsort(jnp.logical_not(m1), stable=True).astype(jnp.int32)
    idx2 = jnp.argsort(jnp.logical_not(m2), stable=True).astype(jnp.int32)
    p = jnp.arange(CAPR, dtype=jnp.int32)
    src1 = idx1[jnp.minimum(p, B - 1)]
    src2 = idx2[jnp.clip(p - c1a, 0, B - 1)]
    idx_arr = jnp.where(p < c1, src1,
                        jnp.where((p >= c1a) & (p < c1a + c2), src2, 0))
    scal = jnp.stack([nb1, nbtot]).astype(jnp.int32)
    cnt16 = jnp.concatenate([c1[None], c2[None], c1a.astype(jnp.int32)[None],
                             jnp.zeros((13,), jnp.int32)])
    rank1 = jnp.cumsum(m1).astype(jnp.int32) - 1
    rank2 = jnp.cumsum(m2).astype(jnp.int32) - 1
    pos_flat = jnp.where(m1, rank1,
                         jnp.where(m2, c1a + rank2, 0)).astype(jnp.int32)
    sel = (tf > 0).astype(jnp.int32)
    base = jax.random.uniform(jax.random.key(1), (B,), dtype=jnp.float32)

    g1, g2, g3 = _sc_gather(l_ty, l_ey, l_y, idx_arr, cnt16)

    h1 = pl.pallas_call(
        _k_layer1,
        grid_spec=pltpu.PrefetchScalarGridSpec(
            num_scalar_prefetch=1,
            grid=(NB, MBLK),
            in_specs=[
                pl.BlockSpec((BM, H), lambda n, m, sc: (_mclamp(m, sc), 0)),
                pl.BlockSpec((BM, H), lambda n, m, sc: (_mclamp(m, sc), 0)),
                pl.BlockSpec((BM, H), lambda n, m, sc: (_mclamp(m, sc), 0)),
                pl.BlockSpec((1, DIN, BN), lambda n, m, sc: (_lvl(m, sc), 0, n)),
                pl.BlockSpec((1, 1, BN), lambda n, m, sc: (_lvl(m, sc), 0, n)),
            ],
            out_specs=pl.BlockSpec(
                (BM, BN), lambda n, m, sc: (_mclamp(m, sc), n)),
        ),
        out_shape=jax.ShapeDtypeStruct((CAPR, DOUT), BF),
    )(scal, g1, g2, g3, W1s, b1s)

    out4 = pl.pallas_call(
        _k_layers234,
        grid_spec=pltpu.PrefetchScalarGridSpec(
            num_scalar_prefetch=1,
            grid=(MBLK,),
            in_specs=[
                pl.BlockSpec((BM, DOUT), lambda m, sc: (_mclamp(m, sc), 0)),
                pl.BlockSpec((1, DOUT, DOUT), lambda m, sc: (_lvl(m, sc), 0, 0)),
                pl.BlockSpec((1, 1, DOUT), lambda m, sc: (_lvl(m, sc), 0, 0)),
                pl.BlockSpec((1, DOUT, DOUT), lambda m, sc: (_lvl(m, sc), 0, 0)),
                pl.BlockSpec((1, 1, DOUT), lambda m, sc: (_lvl(m, sc), 0, 0)),
                pl.BlockSpec((1, 1, DOUT), lambda m, sc: (_lvl(m, sc), 0, 0)),
                pl.BlockSpec((1, 1, BM), lambda m, sc: (_lvl(m, sc), 0, 0)),
            ],
            out_specs=pl.BlockSpec(
                (1, 1, BM), lambda m, sc: (_mclamp(m, sc), 0, 0)),
        ),
        out_shape=jax.ShapeDtypeStruct((MBLK, 1, BM), jnp.float32),
    )(scal, h1, W2s, b2s, W3s, b3s, W4s, b4s)

    return _sc_merge(out4.reshape(CAPR), pos_flat, sel, base)


# speedup vs baseline: 1.0112x; 1.0112x over previous
import functools

import jax
import jax.numpy as jnp
from jax import lax
from jax.experimental import pallas as pl
from jax.experimental.pallas import tpu as pltpu
from jax.experimental.pallas import tpu_sc as plsc

B = 8192
H = 2048
DIN = 3 * H
DOUT = 2048
BM = 256
BN = 1024
NB = DOUT // BN
CAPR = B + 2 * BM
MBLK = CAPR // BM
BF = jnp.bfloat16

NC = 2
NS = 16
NW = NC * NS
CH = 16
SLICE = B // NW

_mesh = plsc.VectorSubcoreMesh(core_axis_name="c", subcore_axis_name="s")


def _leaky(x):
    return jnp.where(x >= 0, x, 0.01 * x)


@functools.partial(
    pl.kernel,
    out_type=[jax.ShapeDtypeStruct((CAPR, H), jnp.float32)] * 3,
    mesh=_mesh,
    compiler_params=pltpu.CompilerParams(needs_layout_passes=False),
    scratch_types=[
        pltpu.VMEM((CH,), jnp.int32),
        pltpu.VMEM((16,), jnp.int32),
        pltpu.VMEM((CH, H), jnp.float32),
        pltpu.VMEM((CH, H), jnp.float32),
        pltpu.SemaphoreType.DMA,
        pltpu.SemaphoreType.DMA,
    ],
)
def _sc_gather(t1, t2, t3, idx, cnt, g1, g2, g3,
               idx_v, cnt_v, ra, rb, sa, sb):
    wid = lax.axis_index("s") * NC + lax.axis_index("c")
    pltpu.sync_copy(cnt, cnt_v)
    cv = cnt_v[...]
    c1s = cv[0]
    c2s = cv[1]
    c1as = cv[2]

    def _chunk(p0):
        p0 = pl.multiple_of(p0, CH)
        pltpu.sync_copy(idx.at[pl.ds(p0, CH)], idx_v)
        cpa = pltpu.async_copy(t1.at[idx_v], ra, sa)
        cpb = pltpu.async_copy(t2.at[idx_v], rb, sb)
        cpa.wait()
        pltpu.sync_copy(ra, g1.at[pl.ds(p0, CH)])
        cpa2 = pltpu.async_copy(t3.at[idx_v], ra, sa)
        cpb.wait()
        pltpu.sync_copy(rb, g2.at[pl.ds(p0, CH)])
        cpa2.wait()
        pltpu.sync_copy(ra, g3.at[pl.ds(p0, CH)])

    c1ch = lax.shift_right_logical(c1s + (CH - 1), 4)
    c2ch = lax.shift_right_logical(c2s + (CH - 1), 4)
    n1w = lax.shift_right_logical(c1ch - wid + (NW - 1), 5)
    n2w = lax.shift_right_logical(c2ch - wid + (NW - 1), 5)

    def _body1(i, carry):
        _chunk((i * NW + wid) * CH)
        return carry

    def _body2(i, carry):
        _chunk(c1as + (i * NW + wid) * CH)
        return carry

    lax.fori_loop(0, n1w, _body1, 0)
    lax.fori_loop(0, n2w, _body2, 0)


@functools.partial(
    pl.kernel,
    out_type=jax.ShapeDtypeStruct((B,), jnp.float32),
    mesh=_mesh,
    compiler_params=pltpu.CompilerParams(needs_layout_passes=False),
    scratch_types=[
        pltpu.VMEM((CAPR,), jnp.float32),
        pltpu.VMEM((SLICE,), jnp.int32),
        pltpu.VMEM((SLICE,), jnp.int32),
        pltpu.VMEM((SLICE,), jnp.float32),
        pltpu.VMEM((SLICE,), jnp.float32),
    ],
)
def _sc_merge(v, pos, sel, bas, res, v_ts, pos_v, sel_v, bas_v, out_v):
    wid = lax.axis_index("s") * NC + lax.axis_index("c")
    o = wid * SLICE
    pltpu.sync_copy(v, v_ts)
    pltpu.sync_copy(pos.at[pl.ds(o, SLICE)], pos_v)
    pltpu.sync_copy(sel.at[pl.ds(o, SLICE)], sel_v)
    pltpu.sync_copy(bas.at[pl.ds(o, SLICE)], bas_v)
    for j in range(SLICE // CH):
        pv = pos_v[pl.ds(j * CH, CH)]
        col = plsc.load_gather(v_ts, [pv])
        sv = sel_v[pl.ds(j * CH, CH)]
        bv = bas_v[pl.ds(j * CH, CH)]
        out_v[pl.ds(j * CH, CH)] = jnp.where(sv > 0, col, bv)
    pltpu.sync_copy(out_v, res.at[pl.ds(o, SLICE)])


def _k_layer1(sc_ref, x1, x2, x3, w, b, out):
    m = pl.program_id(1)

    @pl.when(m < sc_ref[1])
    def _():
        xa = x1[...].astype(BF)
        xb = x2[...].astype(BF)
        xc = x3[...].astype(BF)
        W = w[0]
        acc = jnp.dot(xa, W[0:H], preferred_element_type=jnp.float32)
        acc += jnp.dot(xb, W[H:2 * H], preferred_element_type=jnp.float32)
        acc += jnp.dot(xc, W[2 * H:3 * H], preferred_element_type=jnp.float32)
        acc += b[0]
        out[...] = _leaky(acc).astype(BF)


def _k_layers234(sc_ref, h1, w2, b2, w3, b3, w4, b4, out):
    m = pl.program_id(0)

    @pl.when(m < sc_ref[1])
    def _():
        x = h1[...]
        h2 = _leaky(jnp.dot(x, w2[0], preferred_element_type=jnp.float32)
                    + b2[0]).astype(BF)
        h3 = _leaky(jnp.dot(h2, w3[0], preferred_element_type=jnp.float32)
                    + b3[0])
        col = jnp.sum(h3 * w4[0], axis=1) + b4[0, 0]
        out[...] = col.reshape(1, 1, BM)


def _mclamp(m, sc):
    return jnp.minimum(m, jnp.maximum(sc[1] - 1, 0))


def _lvl(m, sc):
    return jnp.where(_mclamp(m, sc) >= sc[0], 1, 0)


def kernel(l_ty, l_ey, l_y, t,
           W1_l1, b1_l1, W2_l1, b2_l1, W3_l1, b3_l1, W4_l1, b4_l1,
           W1_l2, b1_l2, W2_l2, b2_l2, W3_l2, b3_l2, W4_l2, b4_l2):
    W1s = jnp.stack([W1_l1.T, W1_l2.T]).astype(BF)
    W2s = jnp.stack([W2_l1.T, W2_l2.T]).astype(BF)
    W3s = jnp.stack([W3_l1.T, W3_l2.T]).astype(BF)
    W4s = jnp.stack([W4_l1, W4_l2])
    b1s = jnp.stack([b1_l1, b1_l2])[:, None, :]
    b2s = jnp.stack([b2_l1, b2_l2])[:, None, :]
    b3s = jnp.stack([b3_l1, b3_l2])[:, None, :]
    b4s = jnp.broadcast_to(jnp.stack([b4_l1, b4_l2]).reshape(2, 1, 1),
                           (2, 1, BM))

    tf = t[:, 0]
    m1 = tf == 1
    m2 = tf == 2
    c1 = jnp.sum(m1).astype(jnp.int32)
    c2 = jnp.sum(m2).astype(jnp.int32)
    nb1 = (c1 + BM - 1) // BM
    nbtot = nb1 + (c2 + BM - 1) // BM
    c1a = nb1 * BM
    idx1 = jnp.argsort(jnp.logical_not(m1), stable=True).astype(jnp.int32)
    idx2 = jnp.argsort(jnp.logical_not(m2), stable=True).astype(jnp.int32)
    p = jnp.arange(CAPR, dtype=jnp.int32)
    src1 = idx1[jnp.minimum(p, B - 1)]
    src2 = idx2[jnp.clip(p - c1a, 0, B - 1)]
    idx_arr = jnp.where(p < c1, src1,
                        jnp.where((p >= c1a) & (p < c1a + c2), src2, 0))
    scal = jnp.stack([nb1, nbtot]).astype(jnp.int32)
    cnt16 = jnp.concatenate([c1[None], c2[None], c1a.astype(jnp.int32)[None],
                             jnp.zeros((13,), jnp.int32)])
    rank1 = jnp.cumsum(m1).astype(jnp.int32) - 1
    rank2 = jnp.cumsum(m2).astype(jnp.int32) - 1
    pos_flat = jnp.where(m1, rank1,
                         jnp.where(m2, c1a + rank2, 0)).astype(jnp.int32)
    sel = (tf > 0).astype(jnp.int32)
    base = jax.random.uniform(jax.random.key(1), (B,), dtype=jnp.float32)

    g1, g2, g3 = _sc_gather(l_ty, l_ey, l_y, idx_arr, cnt16)

    h1 = pl.pallas_call(
        _k_layer1,
        grid_spec=pltpu.PrefetchScalarGridSpec(
            num_scalar_prefetch=1,
            grid=(NB, MBLK),
            in_specs=[
                pl.BlockSpec((BM, H), lambda n, m, sc: (_mclamp(m, sc), 0)),
                pl.BlockSpec((BM, H), lambda n, m, sc: (_mclamp(m, sc), 0)),
                pl.BlockSpec((BM, H), lambda n, m, sc: (_mclamp(m, sc), 0)),
                pl.BlockSpec((1, DIN, BN), lambda n, m, sc: (_lvl(m, sc), 0, n)),
                pl.BlockSpec((1, 1, BN), lambda n, m, sc: (_lvl(m, sc), 0, n)),
            ],
            out_specs=pl.BlockSpec(
                (BM, BN), lambda n, m, sc: (_mclamp(m, sc), n)),
        ),
        out_shape=jax.ShapeDtypeStruct((CAPR, DOUT), BF),
    )(scal, g1, g2, g3, W1s, b1s)

    out4 = pl.pallas_call(
        _k_layers234,
        grid_spec=pltpu.PrefetchScalarGridSpec(
            num_scalar_prefetch=1,
            grid=(MBLK,),
            in_specs=[
                pl.BlockSpec((BM, DOUT), lambda m, sc: (_mclamp(m, sc), 0)),
                pl.BlockSpec((1, DOUT, DOUT), lambda m, sc: (_lvl(m, sc), 0, 0)),
                pl.BlockSpec((1, 1, DOUT), lambda m, sc: (_lvl(m, sc), 0, 0)),
                pl.BlockSpec((1, DOUT, DOUT), lambda m, sc: (_lvl(m, sc), 0, 0)),
                pl.BlockSpec((1, 1, DOUT), lambda m, sc: (_lvl(m, sc), 0, 0)),
                pl.BlockSpec((1, 1, DOUT), lambda m, sc: (_lvl(m, sc), 0, 0)),
                pl.BlockSpec((1, 1, BM), lambda m, sc: (_lvl(m, sc), 0, 0)),
            ],
            out_specs=pl.BlockSpec(
                (1, 1, BM), lambda m, sc: (_mclamp(m, sc), 0, 0)),
        ),
        out_shape=jax.ShapeDtypeStruct((MBLK, 1, BM), jnp.float32),
    )(scal, h1, W2s, b2s, W3s, b3s, W4s, b4s)

    return _sc_merge(out4.reshape(CAPR), pos_flat, sel, base)


# speedup vs baseline: 1.0298x; 1.0184x over previous
import functools

import jax
import jax.numpy as jnp
from jax import lax
from jax.experimental import pallas as pl
from jax.experimental.pallas import tpu as pltpu
from jax.experimental.pallas import tpu_sc as plsc

B = 8192
H = 2048
DIN = 3 * H
DOUT = 2048
BM = 256
BN = 1024
NB = DOUT // BN
CAPR = B + 2 * BM
MBLK = CAPR // BM
BF = jnp.bfloat16

NC = 2
NS = 16
NW = NC * NS
CH = 16
SLICE = B // NW

_mesh = plsc.VectorSubcoreMesh(core_axis_name="c", subcore_axis_name="s")


def _leaky(x):
    return jnp.where(x >= 0, x, 0.01 * x)


@functools.partial(
    pl.kernel,
    out_type=[jax.ShapeDtypeStruct((CAPR, H), jnp.float32)] * 3,
    mesh=_mesh,
    compiler_params=pltpu.CompilerParams(needs_layout_passes=False),
    scratch_types=[
        pltpu.VMEM((CH,), jnp.int32),
        pltpu.VMEM((16,), jnp.int32),
        pltpu.VMEM((CH, H), jnp.float32),
        pltpu.VMEM((CH, H), jnp.float32),
        pltpu.SemaphoreType.DMA,
        pltpu.SemaphoreType.DMA,
    ],
)
def _sc_gather(t1, t2, t3, idx, cnt, g1, g2, g3,
               idx_v, cnt_v, ra, rb, sa, sb):
    wid = lax.axis_index("s") * NC + lax.axis_index("c")
    pltpu.sync_copy(cnt, cnt_v)
    cv = cnt_v[...]
    c1s = cv[0]
    c2s = cv[1]
    c1as = cv[2]

    def _chunk(p0):
        p0 = pl.multiple_of(p0, CH)
        pltpu.sync_copy(idx.at[pl.ds(p0, CH)], idx_v)
        cpa = pltpu.async_copy(t1.at[idx_v], ra, sa)
        cpb = pltpu.async_copy(t2.at[idx_v], rb, sb)
        cpa.wait()
        pltpu.sync_copy(ra, g1.at[pl.ds(p0, CH)])
        cpa2 = pltpu.async_copy(t3.at[idx_v], ra, sa)
        cpb.wait()
        pltpu.sync_copy(rb, g2.at[pl.ds(p0, CH)])
        cpa2.wait()
        pltpu.sync_copy(ra, g3.at[pl.ds(p0, CH)])

    c1ch = lax.shift_right_logical(c1s + (CH - 1), 4)
    c2ch = lax.shift_right_logical(c2s + (CH - 1), 4)
    n1w = lax.shift_right_logical(c1ch - wid + (NW - 1), 5)
    n2w = lax.shift_right_logical(c2ch - wid + (NW - 1), 5)

    def _body1(i, carry):
        _chunk((i * NW + wid) * CH)
        return carry

    def _body2(i, carry):
        _chunk(c1as + (i * NW + wid) * CH)
        return carry

    lax.fori_loop(0, n1w, _body1, 0)
    lax.fori_loop(0, n2w, _body2, 0)


@functools.partial(
    pl.kernel,
    out_type=jax.ShapeDtypeStruct((B,), jnp.float32),
    mesh=_mesh,
    compiler_params=pltpu.CompilerParams(needs_layout_passes=False),
    scratch_types=[
        pltpu.VMEM((CAPR,), jnp.float32),
        pltpu.VMEM((SLICE,), jnp.int32),
        pltpu.VMEM((SLICE,), jnp.int32),
        pltpu.VMEM((SLICE,), jnp.float32),
        pltpu.VMEM((SLICE,), jnp.float32),
    ],
)
def _sc_merge(v, pos, sel, bas, res, v_ts, pos_v, sel_v, bas_v, out_v):
    wid = lax.axis_index("s") * NC + lax.axis_index("c")
    o = wid * SLICE
    pltpu.sync_copy(v, v_ts)
    pltpu.sync_copy(pos.at[pl.ds(o, SLICE)], pos_v)
    pltpu.sync_copy(sel.at[pl.ds(o, SLICE)], sel_v)
    pltpu.sync_copy(bas.at[pl.ds(o, SLICE)], bas_v)
    for j in range(SLICE // CH):
        pv = pos_v[pl.ds(j * CH, CH)]
        col = plsc.load_gather(v_ts, [pv])
        sv = sel_v[pl.ds(j * CH, CH)]
        bv = bas_v[pl.ds(j * CH, CH)]
        out_v[pl.ds(j * CH, CH)] = jnp.where(sv > 0, col, bv)
    pltpu.sync_copy(out_v, res.at[pl.ds(o, SLICE)])


def _k_layer1(sc_ref, x1, x2, x3, w, b, out):
    m = pl.program_id(1)

    @pl.when(m < sc_ref[1])
    def _():
        xa = x1[...].astype(BF)
        xb = x2[...].astype(BF)
        xc = x3[...].astype(BF)
        W = w[0]
        acc = jnp.dot(xa, W[0:H], preferred_element_type=jnp.float32)
        acc += jnp.dot(xb, W[H:2 * H], preferred_element_type=jnp.float32)
        acc += jnp.dot(xc, W[2 * H:3 * H], preferred_element_type=jnp.float32)
        acc += b[0]
        out[...] = _leaky(acc).astype(BF)


def _k_layers234(sc_ref, h1, w2, b2, w3, b3, w4, b4, out):
    m = pl.program_id(0)

    @pl.when(m < sc_ref[1])
    def _():
        x = h1[...]
        h2 = _leaky(jnp.dot(x, w2[0], preferred_element_type=jnp.float32)
                    + b2[0]).astype(BF)
        h3 = _leaky(jnp.dot(h2, w3[0], preferred_element_type=jnp.float32)
                    + b3[0])
        col = jnp.sum(h3 * w4[0], axis=1) + b4[0, 0]
        out[...] = col.reshape(1, 1, BM)


def _mclamp(m, sc):
    return jnp.minimum(m, jnp.maximum(sc[1] - 1, 0))


def _lvl(m, sc):
    return jnp.where(_mclamp(m, sc) >= sc[0], 1, 0)


def kernel(l_ty, l_ey, l_y, t,
           W1_l1, b1_l1, W2_l1, b2_l1, W3_l1, b3_l1, W4_l1, b4_l1,
           W1_l2, b1_l2, W2_l2, b2_l2, W3_l2, b3_l2, W4_l2, b4_l2):
    W1s = jnp.stack([W1_l1.T, W1_l2.T]).astype(BF)
    W2s = jnp.stack([W2_l1.T, W2_l2.T]).astype(BF)
    W3s = jnp.stack([W3_l1.T, W3_l2.T]).astype(BF)
    W4s = jnp.stack([W4_l1, W4_l2])
    b1s = jnp.stack([b1_l1, b1_l2])[:, None, :]
    b2s = jnp.stack([b2_l1, b2_l2])[:, None, :]
    b3s = jnp.stack([b3_l1, b3_l2])[:, None, :]
    b4s = jnp.broadcast_to(jnp.stack([b4_l1, b4_l2]).reshape(2, 1, 1),
                           (2, 1, BM))

    tf = t[:, 0]
    m1 = tf == 1
    m2 = tf == 2
    c1 = jnp.sum(m1).astype(jnp.int32)
    c2 = jnp.sum(m2).astype(jnp.int32)
    nb1 = (c1 + BM - 1) // BM
    nbtot = nb1 + (c2 + BM - 1) // BM
    c1a = nb1 * BM
    scal = jnp.stack([nb1, nbtot]).astype(jnp.int32)
    cnt16 = jnp.concatenate([c1[None], c2[None], c1a.astype(jnp.int32)[None],
                             jnp.zeros((13,), jnp.int32)])
    rank1 = jnp.cumsum(m1).astype(jnp.int32) - 1
    rank2 = jnp.cumsum(m2).astype(jnp.int32) - 1
    pos_flat = jnp.where(m1, rank1,
                         jnp.where(m2, c1a + rank2, 0)).astype(jnp.int32)
    sel = (tf > 0).astype(jnp.int32)
    scat_idx = jnp.where(tf > 0, pos_flat, CAPR)
    idx_arr = jnp.zeros((CAPR,), jnp.int32).at[scat_idx].set(
        jnp.arange(B, dtype=jnp.int32), mode="drop", unique_indices=True)
    base = jax.random.uniform(jax.random.key(1), (B,), dtype=jnp.float32)

    g1, g2, g3 = _sc_gather(l_ty, l_ey, l_y, idx_arr, cnt16)

    h1 = pl.pallas_call(
        _k_layer1,
        grid_spec=pltpu.PrefetchScalarGridSpec(
            num_scalar_prefetch=1,
            grid=(NB, MBLK),
            in_specs=[
                pl.BlockSpec((BM, H), lambda n, m, sc: (_mclamp(m, sc), 0)),
                pl.BlockSpec((BM, H), lambda n, m, sc: (_mclamp(m, sc), 0)),
                pl.BlockSpec((BM, H), lambda n, m, sc: (_mclamp(m, sc), 0)),
                pl.BlockSpec((1, DIN, BN), lambda n, m, sc: (_lvl(m, sc), 0, n)),
                pl.BlockSpec((1, 1, BN), lambda n, m, sc: (_lvl(m, sc), 0, n)),
            ],
            out_specs=pl.BlockSpec(
                (BM, BN), lambda n, m, sc: (_mclamp(m, sc), n)),
        ),
        out_shape=jax.ShapeDtypeStruct((CAPR, DOUT), BF),
    )(scal, g1, g2, g3, W1s, b1s)

    out4 = pl.pallas_call(
        _k_layers234,
        grid_spec=pltpu.PrefetchScalarGridSpec(
            num_scalar_prefetch=1,
            grid=(MBLK,),
            in_specs=[
                pl.BlockSpec((BM, DOUT), lambda m, sc: (_mclamp(m, sc), 0)),
                pl.BlockSpec((1, DOUT, DOUT), lambda m, sc: (_lvl(m, sc), 0, 0)),
                pl.BlockSpec((1, 1, DOUT), lambda m, sc: (_lvl(m, sc), 0, 0)),
                pl.BlockSpec((1, DOUT, DOUT), lambda m, sc: (_lvl(m, sc), 0, 0)),
                pl.BlockSpec((1, 1, DOUT), lambda m, sc: (_lvl(m, sc), 0, 0)),
                pl.BlockSpec((1, 1, DOUT), lambda m, sc: (_lvl(m, sc), 0, 0)),
                pl.BlockSpec((1, 1, BM), lambda m, sc: (_lvl(m, sc), 0, 0)),
            ],
            out_specs=pl.BlockSpec(
                (1, 1, BM), lambda m, sc: (_mclamp(m, sc), 0, 0)),
        ),
        out_shape=jax.ShapeDtypeStruct((MBLK, 1, BM), jnp.float32),
    )(scal, h1, W2s, b2s, W3s, b3s, W4s, b4s)

    return _sc_merge(out4.reshape(CAPR), pos_flat, sel, base)


# speedup vs baseline: 1.0784x; 1.0472x over previous
import functools

import jax
import jax.numpy as jnp
from jax import lax
from jax.experimental import pallas as pl
from jax.experimental.pallas import tpu as pltpu
from jax.experimental.pallas import tpu_sc as plsc

B = 8192
H = 2048
DIN = 3 * H
DOUT = 2048
BM = 256
BN = 1024
NB = DOUT // BN
MB = B // BM
BF = jnp.bfloat16

NC = 2
NS = 16
NW = NC * NS
CH = 16
SLICE = B // NW

_mesh = plsc.VectorSubcoreMesh(core_axis_name="c", subcore_axis_name="s")


def _leaky(x):
    return jnp.where(x >= 0, x, 0.01 * x)


@functools.partial(
    pl.kernel,
    out_type=[jax.ShapeDtypeStruct((B, H), jnp.float32)] * 3,
    mesh=_mesh,
    compiler_params=pltpu.CompilerParams(needs_layout_passes=False),
    scratch_types=[
        pltpu.VMEM((CH,), jnp.int32),
        pltpu.VMEM((16,), jnp.int32),
        pltpu.VMEM((CH, H), jnp.float32),
        pltpu.VMEM((CH, H), jnp.float32),
        pltpu.SemaphoreType.DMA,
        pltpu.SemaphoreType.DMA,
    ],
)
def _sc_gather(t1, t2, t3, idx, cnt, g1, g2, g3,
               idx_v, cnt_v, ra, rb, sa, sb):
    wid = lax.axis_index("s") * NC + lax.axis_index("c")
    pltpu.sync_copy(cnt, cnt_v)
    cs = cnt_v[...][0]

    def _chunk(p0):
        p0 = pl.multiple_of(p0, CH)
        pltpu.sync_copy(idx.at[pl.ds(p0, CH)], idx_v)
        cpa = pltpu.async_copy(t1.at[idx_v], ra, sa)
        cpb = pltpu.async_copy(t2.at[idx_v], rb, sb)
        cpa.wait()
        pltpu.sync_copy(ra, g1.at[pl.ds(p0, CH)])
        cpa2 = pltpu.async_copy(t3.at[idx_v], ra, sa)
        cpb.wait()
        pltpu.sync_copy(rb, g2.at[pl.ds(p0, CH)])
        cpa2.wait()
        pltpu.sync_copy(ra, g3.at[pl.ds(p0, CH)])

    cch = lax.shift_right_logical(cs + (CH - 1), 4)
    nwk = lax.shift_right_logical(cch - wid + (NW - 1), 5)

    def _body(i, carry):
        _chunk((i * NW + wid) * CH)
        return carry

    lax.fori_loop(0, nwk, _body, 0)


@functools.partial(
    pl.kernel,
    out_type=jax.ShapeDtypeStruct((B,), jnp.float32),
    mesh=_mesh,
    compiler_params=pltpu.CompilerParams(needs_layout_passes=False),
    scratch_types=[
        pltpu.VMEM((2 * B,), jnp.float32),
        pltpu.VMEM((SLICE,), jnp.int32),
        pltpu.VMEM((SLICE,), jnp.int32),
        pltpu.VMEM((SLICE,), jnp.float32),
        pltpu.VMEM((SLICE,), jnp.float32),
    ],
)
def _sc_merge(v1, v2, pos, sel, bas, res, v_ts, pos_v, sel_v, bas_v, out_v):
    wid = lax.axis_index("s") * NC + lax.axis_index("c")
    o = wid * SLICE
    pltpu.sync_copy(v1, v_ts.at[pl.ds(0, B)])
    pltpu.sync_copy(v2, v_ts.at[pl.ds(B, B)])
    pltpu.sync_copy(pos.at[pl.ds(o, SLICE)], pos_v)
    pltpu.sync_copy(sel.at[pl.ds(o, SLICE)], sel_v)
    pltpu.sync_copy(bas.at[pl.ds(o, SLICE)], bas_v)
    for j in range(SLICE // CH):
        pv = pos_v[pl.ds(j * CH, CH)]
        col = plsc.load_gather(v_ts, [pv])
        sv = sel_v[pl.ds(j * CH, CH)]
        bv = bas_v[pl.ds(j * CH, CH)]
        out_v[pl.ds(j * CH, CH)] = jnp.where(sv > 0, col, bv)
    pltpu.sync_copy(out_v, res.at[pl.ds(o, SLICE)])


def _k_layer1(sc_ref, x1, x2, x3, w, b, out):
    m = pl.program_id(1)

    @pl.when(m < sc_ref[0])
    def _():
        xa = x1[...].astype(BF)
        xb = x2[...].astype(BF)
        xc = x3[...].astype(BF)
        W = w[...]
        acc = jnp.dot(xa, W[0:H], preferred_element_type=jnp.float32)
        acc += jnp.dot(xb, W[H:2 * H], preferred_element_type=jnp.float32)
        acc += jnp.dot(xc, W[2 * H:3 * H], preferred_element_type=jnp.float32)
        acc += b[...]
        out[...] = _leaky(acc).astype(BF)


def _k_layers234(sc_ref, h1, w2, b2, w3, b3, w4, b4, out):
    m = pl.program_id(0)

    @pl.when(m < sc_ref[0])
    def _():
        x = h1[...]
        h2 = _leaky(jnp.dot(x, w2[...], preferred_element_type=jnp.float32)
                    + b2[...]).astype(BF)
        h3 = _leaky(jnp.dot(h2, w3[...], preferred_element_type=jnp.float32)
                    + b3[...])
        col = jnp.sum(h3 * w4[...], axis=1) + b4[0]
        out[...] = col.reshape(1, 1, BM)


def _mclamp(m, sc):
    return jnp.minimum(m, jnp.maximum(sc[0] - 1, 0))


def _level_mlp(g1, g2, g3, scal, W1t, b1, W2t, b2, W3t, b3, w4, b4bc):
    h1 = pl.pallas_call(
        _k_layer1,
        grid_spec=pltpu.PrefetchScalarGridSpec(
            num_scalar_prefetch=1,
            grid=(NB, MB),
            in_specs=[
                pl.BlockSpec((BM, H), lambda n, m, sc: (_mclamp(m, sc), 0)),
                pl.BlockSpec((BM, H), lambda n, m, sc: (_mclamp(m, sc), 0)),
                pl.BlockSpec((BM, H), lambda n, m, sc: (_mclamp(m, sc), 0)),
                pl.BlockSpec((DIN, BN), lambda n, m, sc: (0, n)),
                pl.BlockSpec((1, BN), lambda n, m, sc: (0, n)),
            ],
            out_specs=pl.BlockSpec(
                (BM, BN), lambda n, m, sc: (_mclamp(m, sc), n)),
        ),
        out_shape=jax.ShapeDtypeStruct((B, DOUT), BF),
    )(scal, g1, g2, g3, W1t, b1)

    out4 = pl.pallas_call(
        _k_layers234,
        grid_spec=pltpu.PrefetchScalarGridSpec(
            num_scalar_prefetch=1,
            grid=(MB,),
            in_specs=[
                pl.BlockSpec((BM, DOUT), lambda m, sc: (_mclamp(m, sc), 0)),
                pl.BlockSpec((DOUT, DOUT), lambda m, sc: (0, 0)),
                pl.BlockSpec((1, DOUT), lambda m, sc: (0, 0)),
                pl.BlockSpec((DOUT, DOUT), lambda m, sc: (0, 0)),
                pl.BlockSpec((1, DOUT), lambda m, sc: (0, 0)),
                pl.BlockSpec((1, DOUT), lambda m, sc: (0, 0)),
                pl.BlockSpec((1, BM), lambda m, sc: (0, 0)),
            ],
            out_specs=pl.BlockSpec(
                (1, 1, BM), lambda m, sc: (_mclamp(m, sc), 0, 0)),
        ),
        out_shape=jax.ShapeDtypeStruct((MB, 1, BM), jnp.float32),
    )(scal, h1, W2t, b2, W3t, b3, w4, b4bc)
    return out4.reshape(B)


def kernel(l_ty, l_ey, l_y, t,
           W1_l1, b1_l1, W2_l1, b2_l1, W3_l1, b3_l1, W4_l1, b4_l1,
           W1_l2, b1_l2, W2_l2, b2_l2, W3_l2, b3_l2, W4_l2, b4_l2):
    tf = t[:, 0]
    base = jax.random.uniform(jax.random.key(1), (B,), dtype=jnp.float32)
    sel = (tf > 0).astype(jnp.int32)
    arange_b = jnp.arange(B, dtype=jnp.int32)

    vs = []
    ranks = []
    masks = []
    params = (
        (W1_l1, b1_l1, W2_l1, b2_l1, W3_l1, b3_l1, W4_l1, b4_l1),
        (W1_l2, b1_l2, W2_l2, b2_l2, W3_l2, b3_l2, W4_l2, b4_l2),
    )
    for lvl, (W1, b1, W2, b2, W3, b3, W4, b4) in zip((1, 2), params):
        mk = tf == lvl
        cnt = jnp.sum(mk).astype(jnp.int32)
        rank = jnp.cumsum(mk).astype(jnp.int32) - 1
        idx_arr = jnp.zeros((B,), jnp.int32).at[
            jnp.where(mk, rank, B)].set(arange_b, mode="drop",
                                        unique_indices=True)
        cnt16 = jnp.concatenate([cnt[None], jnp.zeros((15,), jnp.int32)])
        scal = ((cnt + BM - 1) // BM).astype(jnp.int32)[None]

        g1, g2, g3 = _sc_gather(l_ty, l_ey, l_y, idx_arr, cnt16)

        W1t = W1.T.astype(BF)
        W2t = W2.T.astype(BF)
        W3t = W3.T.astype(BF)
        b4bc = jnp.broadcast_to(b4.reshape(1, 1), (1, BM))
        vs.append(_level_mlp(g1, g2, g3, scal,
                             W1t, b1[None, :], W2t, b2[None, :],
                             W3t, b3[None, :], W4, b4bc))
        ranks.append(rank)
        masks.append(mk)

    pos_flat = jnp.where(masks[0], ranks[0],
                         jnp.where(masks[1], B + ranks[1], 0)).astype(jnp.int32)
    return _sc_merge(vs[0], vs[1], pos_flat, sel, base)


# speedup vs baseline: 1.1436x; 1.0605x over previous
import functools

import jax
import jax.numpy as jnp
from jax import lax
from jax.experimental import pallas as pl
from jax.experimental.pallas import tpu as pltpu
from jax.experimental.pallas import tpu_sc as plsc

B = 8192
H = 2048
DIN = 3 * H
DOUT = 2048
BM = 256
BN = 2048
NB = DOUT // BN
MB = B // BM
BF = jnp.bfloat16

NC = 2
NS = 16
NW = NC * NS
CH = 16
SLICE = B // NW

_mesh = plsc.VectorSubcoreMesh(core_axis_name="c", subcore_axis_name="s")


def _leaky(x):
    return jnp.where(x >= 0, x, 0.01 * x)


@functools.partial(
    pl.kernel,
    out_type=[jax.ShapeDtypeStruct((B, H), jnp.float32)] * 3,
    mesh=_mesh,
    compiler_params=pltpu.CompilerParams(needs_layout_passes=False),
    scratch_types=[
        pltpu.VMEM((CH,), jnp.int32),
        pltpu.VMEM((16,), jnp.int32),
        pltpu.VMEM((CH, H), jnp.float32),
        pltpu.VMEM((CH, H), jnp.float32),
        pltpu.SemaphoreType.DMA,
        pltpu.SemaphoreType.DMA,
    ],
)
def _sc_gather(t1, t2, t3, idx, cnt, g1, g2, g3,
               idx_v, cnt_v, ra, rb, sa, sb):
    wid = lax.axis_index("s") * NC + lax.axis_index("c")
    pltpu.sync_copy(cnt, cnt_v)
    cs = cnt_v[...][0]

    def _chunk(p0):
        p0 = pl.multiple_of(p0, CH)
        pltpu.sync_copy(idx.at[pl.ds(p0, CH)], idx_v)
        cpa = pltpu.async_copy(t1.at[idx_v], ra, sa)
        cpb = pltpu.async_copy(t2.at[idx_v], rb, sb)
        cpa.wait()
        pltpu.sync_copy(ra, g1.at[pl.ds(p0, CH)])
        cpa2 = pltpu.async_copy(t3.at[idx_v], ra, sa)
        cpb.wait()
        pltpu.sync_copy(rb, g2.at[pl.ds(p0, CH)])
        cpa2.wait()
        pltpu.sync_copy(ra, g3.at[pl.ds(p0, CH)])

    cch = lax.shift_right_logical(cs + (CH - 1), 4)
    nwk = lax.shift_right_logical(cch - wid + (NW - 1), 5)

    def _body(i, carry):
        _chunk((i * NW + wid) * CH)
        return carry

    lax.fori_loop(0, nwk, _body, 0)


@functools.partial(
    pl.kernel,
    out_type=jax.ShapeDtypeStruct((B,), jnp.float32),
    mesh=_mesh,
    compiler_params=pltpu.CompilerParams(needs_layout_passes=False),
    scratch_types=[
        pltpu.VMEM((2 * B,), jnp.float32),
        pltpu.VMEM((SLICE,), jnp.int32),
        pltpu.VMEM((SLICE,), jnp.int32),
        pltpu.VMEM((SLICE,), jnp.float32),
        pltpu.VMEM((SLICE,), jnp.float32),
    ],
)
def _sc_merge(v1, v2, pos, sel, bas, res, v_ts, pos_v, sel_v, bas_v, out_v):
    wid = lax.axis_index("s") * NC + lax.axis_index("c")
    o = wid * SLICE
    pltpu.sync_copy(v1, v_ts.at[pl.ds(0, B)])
    pltpu.sync_copy(v2, v_ts.at[pl.ds(B, B)])
    pltpu.sync_copy(pos.at[pl.ds(o, SLICE)], pos_v)
    pltpu.sync_copy(sel.at[pl.ds(o, SLICE)], sel_v)
    pltpu.sync_copy(bas.at[pl.ds(o, SLICE)], bas_v)
    for j in range(SLICE // CH):
        pv = pos_v[pl.ds(j * CH, CH)]
        col = plsc.load_gather(v_ts, [pv])
        sv = sel_v[pl.ds(j * CH, CH)]
        bv = bas_v[pl.ds(j * CH, CH)]
        out_v[pl.ds(j * CH, CH)] = jnp.where(sv > 0, col, bv)
    pltpu.sync_copy(out_v, res.at[pl.ds(o, SLICE)])


def _k_layer1(sc_ref, x1, x2, x3, w, b, out):
    m = pl.program_id(1)

    @pl.when(m < sc_ref[0])
    def _():
        xa = x1[...].astype(BF)
        xb = x2[...].astype(BF)
        xc = x3[...].astype(BF)
        W = w[...]
        acc = jnp.dot(xa, W[0:H], preferred_element_type=jnp.float32)
        acc += jnp.dot(xb, W[H:2 * H], preferred_element_type=jnp.float32)
        acc += jnp.dot(xc, W[2 * H:3 * H], preferred_element_type=jnp.float32)
        acc += b[...]
        out[...] = _leaky(acc).astype(BF)


def _k_layers234(sc_ref, h1, w2, b2, w3, b3, w4, b4, out):
    m = pl.program_id(0)

    @pl.when(m < sc_ref[0])
    def _():
        x = h1[...]
        h2 = _leaky(jnp.dot(x, w2[...], preferred_element_type=jnp.float32)
                    + b2[...]).astype(BF)
        h3 = _leaky(jnp.dot(h2, w3[...], preferred_element_type=jnp.float32)
                    + b3[...])
        col = jnp.sum(h3 * w4[...], axis=1) + b4[0]
        out[...] = col.reshape(1, 1, BM)


def _mclamp(m, sc):
    return jnp.minimum(m, jnp.maximum(sc[0] - 1, 0))


def _level_mlp(g1, g2, g3, scal, W1t, b1, W2t, b2, W3t, b3, w4, b4bc):
    h1 = pl.pallas_call(
        _k_layer1,
        grid_spec=pltpu.PrefetchScalarGridSpec(
            num_scalar_prefetch=1,
            grid=(NB, MB),
            in_specs=[
                pl.BlockSpec((BM, H), lambda n, m, sc: (_mclamp(m, sc), 0)),
                pl.BlockSpec((BM, H), lambda n, m, sc: (_mclamp(m, sc), 0)),
                pl.BlockSpec((BM, H), lambda n, m, sc: (_mclamp(m, sc), 0)),
                pl.BlockSpec((DIN, BN), lambda n, m, sc: (0, n)),
                pl.BlockSpec((1, BN), lambda n, m, sc: (0, n)),
            ],
            out_specs=pl.BlockSpec(
                (BM, BN), lambda n, m, sc: (_mclamp(m, sc), n)),
        ),
        out_shape=jax.ShapeDtypeStruct((B, DOUT), BF),
    )(scal, g1, g2, g3, W1t, b1)

    out4 = pl.pallas_call(
        _k_layers234,
        grid_spec=pltpu.PrefetchScalarGridSpec(
            num_scalar_prefetch=1,
            grid=(MB,),
            in_specs=[
                pl.BlockSpec((BM, DOUT), lambda m, sc: (_mclamp(m, sc), 0)),
                pl.BlockSpec((DOUT, DOUT), lambda m, sc: (0, 0)),
                pl.BlockSpec((1, DOUT), lambda m, sc: (0, 0)),
                pl.BlockSpec((DOUT, DOUT), lambda m, sc: (0, 0)),
                pl.BlockSpec((1, DOUT), lambda m, sc: (0, 0)),
                pl.BlockSpec((1, DOUT), lambda m, sc: (0, 0)),
                pl.BlockSpec((1, BM), lambda m, sc: (0, 0)),
            ],
            out_specs=pl.BlockSpec(
                (1, 1, BM), lambda m, sc: (_mclamp(m, sc), 0, 0)),
        ),
        out_shape=jax.ShapeDtypeStruct((MB, 1, BM), jnp.float32),
    )(scal, h1, W2t, b2, W3t, b3, w4, b4bc)
    return out4.reshape(B)


def kernel(l_ty, l_ey, l_y, t,
           W1_l1, b1_l1, W2_l1, b2_l1, W3_l1, b3_l1, W4_l1, b4_l1,
           W1_l2, b1_l2, W2_l2, b2_l2, W3_l2, b3_l2, W4_l2, b4_l2):
    tf = t[:, 0]
    base = jax.random.uniform(jax.random.key(1), (B,), dtype=jnp.float32)
    sel = (tf > 0).astype(jnp.int32)
    arange_b = jnp.arange(B, dtype=jnp.int32)

    m1 = tf == 1
    m2 = tf == 2
    enc = jnp.stack([m1, m2]).astype(jnp.float32).reshape(2, 64, 128)
    tri_in = jnp.tril(jnp.ones((128, 128), jnp.float32)).T
    within = jnp.einsum("lrc,ck->lrk", enc, tri_in,
                        preferred_element_type=jnp.float32)
    rowtot = within[:, :, 127]
    offs = jnp.einsum("lr,rk->lk", rowtot,
                      jnp.triu(jnp.ones((64, 64), jnp.float32), 1),
                      preferred_element_type=jnp.float32)
    cum = (within + offs[:, :, None]).reshape(2, B).astype(jnp.int32)
    rank1 = cum[0] - 1
    rank2 = cum[1] - 1
    c1 = cum[0, B - 1]
    c2 = cum[1, B - 1]
    pos_flat = jnp.where(m1, rank1,
                         jnp.where(m2, B + rank2, 0)).astype(jnp.int32)
    idx_comb = jnp.zeros((2 * B,), jnp.int32).at[
        jnp.where(tf > 0, pos_flat, 2 * B)].set(arange_b, mode="drop",
                                                unique_indices=True)

    vs = []
    params = (
        (c1, W1_l1, b1_l1, W2_l1, b2_l1, W3_l1, b3_l1, W4_l1, b4_l1),
        (c2, W1_l2, b1_l2, W2_l2, b2_l2, W3_l2, b3_l2, W4_l2, b4_l2),
    )
    for lvl, (cnt, W1, b1, W2, b2, W3, b3, W4, b4) in zip((1, 2), params):
        idx_arr = lax.slice(idx_comb, ((lvl - 1) * B,), (lvl * B,))
        cnt16 = jnp.concatenate([cnt[None], jnp.zeros((15,), jnp.int32)])
        scal = ((cnt + BM - 1) // BM).astype(jnp.int32)[None]

        g1, g2, g3 = _sc_gather(l_ty, l_ey, l_y, idx_arr, cnt16)

        W1t = W1.T.astype(BF)
        W2t = W2.T.astype(BF)
        W3t = W3.T.astype(BF)
        b4bc = jnp.broadcast_to(b4.reshape(1, 1), (1, BM))
        vs.append(_level_mlp(g1, g2, g3, scal,
                             W1t, b1[None, :], W2t, b2[None, :],
                             W3t, b3[None, :], W4, b4bc))

    return _sc_merge(vs[0], vs[1], pos_flat, sel, base)
